# Initial kernel scaffold; baseline (speedup 1.0000x reference)
import jax
import jax.numpy as jnp
from jax.experimental import pallas as pl


def kernel(user_id, gender, status, regis_date, history, voting, favourite, user_table, gender_table, status_table, rgst_table, hsty_table, vote_table, favr_table):
    raise NotImplementedError("write your pallas kernel here")


# speedup vs baseline: 5.4539x; 5.4539x over previous
import functools

import numpy as np
import jax
import jax.numpy as jnp
from jax import lax
from jax.experimental import pallas as pl
from jax.experimental.pallas import tpu as pltpu
from jax.experimental.pallas import tpu_sc as plsc

B = 16384
EMBED_DIM = 32
NUM_BOUND = 1000
OUT_D = 7 * EMBED_DIM + 4

NC, NS, L = 2, 16, 16
NW = NC * NS
BPW = B // NW
C = 256
NCHUNK = BPW // C

COL_U, COL_G, COL_S, COL_RE = 0, 32, 64, 96
COL_RN = 128
COL_HE, COL_HN = 129, 161
COL_VE, COL_VN = 162, 194
COL_FE, COL_FN = 195, 227

_NORM_DIV = float(np.sqrt(np.float32(1.0 / 12.0 + 1e-7)))

_mesh = plsc.VectorSubcoreMesh(core_axis_name="c", subcore_axis_name="s")


@functools.partial(
    pl.kernel,
    out_type=jax.ShapeDtypeStruct((B, OUT_D), jnp.float32),
    mesh=_mesh,
    compiler_params=pltpu.CompilerParams(use_tc_tiling_on_sc=False,
                                         needs_layout_passes=False),
    scratch_types=[
        pltpu.VMEM((BPW,), jnp.int32),
        pltpu.VMEM((BPW,), jnp.int32),
        pltpu.VMEM((BPW,), jnp.int32),
        pltpu.VMEM((BPW,), jnp.float32),
        pltpu.VMEM((BPW,), jnp.float32),
        pltpu.VMEM((BPW,), jnp.float32),
        pltpu.VMEM((BPW,), jnp.float32),
        pltpu.VMEM((NUM_BOUND,), jnp.float32),
        pltpu.VMEM((BPW,), jnp.int32),
        pltpu.VMEM((BPW,), jnp.int32),
        pltpu.VMEM((BPW,), jnp.int32),
        pltpu.VMEM((BPW,), jnp.int32),
        pltpu.VMEM((C, OUT_D), jnp.float32),
        pltpu.VMEM((C, EMBED_DIM), jnp.float32),
        pltpu.VMEM((C, EMBED_DIM), jnp.float32),
        pltpu.VMEM((C, EMBED_DIM), jnp.float32),
        pltpu.VMEM((C, EMBED_DIM), jnp.float32),
        pltpu.VMEM((C, EMBED_DIM), jnp.float32),
        pltpu.VMEM((C, EMBED_DIM), jnp.float32),
        pltpu.VMEM((C, EMBED_DIM), jnp.float32),
        pltpu.SemaphoreType.DMA,
    ],
)
def _sc_kernel(uid_hbm, gid_hbm, sid_hbm, f0_hbm, f1_hbm, f2_hbm, f3_hbm,
               ut_hbm, gt_hbm, st_hbm, rt_hbm, ht_hbm, vt_hbm, ft_hbm,
               bnd_hbm, out_hbm,
               uid_v, gid_v, sid_v, f0_v, f1_v, f2_v, f3_v, bnd_v,
               b0_v, b1_v, b2_v, b3_v, stage_v,
               ue_v, ge_v, se_v, re_v, he_v, ve_v, fe_v, sem):
    wid = lax.axis_index("s") * NC + lax.axis_index("c")
    base = wid * BPW

    pltpu.sync_copy(uid_hbm.at[pl.ds(base, BPW)], uid_v)
    pltpu.sync_copy(gid_hbm.at[pl.ds(base, BPW)], gid_v)
    pltpu.sync_copy(sid_hbm.at[pl.ds(base, BPW)], sid_v)
    pltpu.sync_copy(f0_hbm.at[pl.ds(base, BPW)], f0_v)
    pltpu.sync_copy(f1_hbm.at[pl.ds(base, BPW)], f1_v)
    pltpu.sync_copy(f2_hbm.at[pl.ds(base, BPW)], f2_v)
    pltpu.sync_copy(f3_hbm.at[pl.ds(base, BPW)], f3_v)
    pltpu.sync_copy(bnd_hbm, bnd_v)

    feats = ((f0_v, b0_v), (f1_v, b1_v), (f2_v, b2_v), (f3_v, b3_v))

    def bidx_body(i, carry):
        sl = pl.ds(i * L, L)
        for fref, bref in feats:
            x = fref[sl]
            c = jnp.clip((x * 999.0).astype(jnp.int32), 0, NUM_BOUND - 1)
            cm1 = jnp.maximum(c - 1, 0)
            cp1 = jnp.minimum(c + 1, NUM_BOUND - 1)
            t0 = (plsc.load_gather(bnd_v, [cm1]) < x).astype(jnp.int32)
            t1 = (plsc.load_gather(bnd_v, [c]) < x).astype(jnp.int32)
            t2 = (plsc.load_gather(bnd_v, [cp1]) < x).astype(jnp.int32)
            bref[sl] = jnp.maximum(c - 1 + t0 + t1 + t2, 0)
        return carry

    lax.fori_loop(0, BPW // L, bidx_body, 0)

    lanes = jnp.arange(L, dtype=jnp.int32)

    for ch in range(NCHUNK):
        rbase = ch * C
        csl = pl.ds(rbase, C)
        copies = [
            pltpu.async_copy(ut_hbm.at[uid_v.at[csl]], ue_v, sem),
            pltpu.async_copy(gt_hbm.at[gid_v.at[csl]], ge_v, sem),
            pltpu.async_copy(st_hbm.at[sid_v.at[csl]], se_v, sem),
            pltpu.async_copy(rt_hbm.at[b0_v.at[csl]], re_v, sem),
            pltpu.async_copy(ht_hbm.at[b1_v.at[csl]], he_v, sem),
            pltpu.async_copy(vt_hbm.at[b2_v.at[csl]], ve_v, sem),
            pltpu.async_copy(ft_hbm.at[b3_v.at[csl]], fe_v, sem),
        ]
        for cp in copies:
            cp.wait()

        fields = ((ue_v, COL_U), (ge_v, COL_G), (se_v, COL_S), (re_v, COL_RE),
                  (he_v, COL_HE), (ve_v, COL_VE), (fe_v, COL_FE))

        def asm_body(i, carry):
            rows = lanes + i * L
            for src_ref, col0 in fields:
                for j in range(EMBED_DIM):
                    jv = jnp.full((L,), j, jnp.int32)
                    x = plsc.load_gather(src_ref, [rows, jv])
                    plsc.store_scatter(stage_v, [rows, jv + col0], x)
            for fref, col in ((f0_v, COL_RN), (f1_v, COL_HN),
                              (f2_v, COL_VN), (f3_v, COL_FN)):
                x = fref[pl.ds(rbase + i * L, L)]
                n = (x - 0.5) / _NORM_DIV
                plsc.store_scatter(
                    stage_v, [rows, jnp.full((L,), col, jnp.int32)], n)
            return carry

        lax.fori_loop(0, C // L, asm_body, 0)
        pltpu.sync_copy(stage_v, out_hbm.at[pl.ds(base + rbase, C), :])


def kernel(user_id, gender, status, regis_date, history, voting, favourite,
           user_table, gender_table, status_table,
           rgst_table, hsty_table, vote_table, favr_table):
    bounds = jnp.linspace(0.0, 1.0, NUM_BOUND)
    return _sc_kernel(
        user_id.astype(jnp.int32), gender.astype(jnp.int32),
        status.astype(jnp.int32), regis_date, history, voting, favourite,
        user_table, gender_table, status_table,
        rgst_table, hsty_table, vote_table, favr_table,
        bounds.astype(jnp.float32))


# speedup vs baseline: 5.4601x; 1.0011x over previous
import functools

import numpy as np
import jax
import jax.numpy as jnp
from jax import lax
from jax.experimental import pallas as pl
from jax.experimental.pallas import tpu as pltpu
from jax.experimental.pallas import tpu_sc as plsc

B = 16384
EMBED_DIM = 32
NUM_BOUND = 1000
OUT_D = 7 * EMBED_DIM + 4

NC, NS, L = 2, 16, 16
NW = NC * NS
BPW = B // NW
C = 256
NCHUNK = BPW // C

COL_U, COL_G, COL_S, COL_RE = 0, 32, 64, 96
COL_RN = 128
COL_HE, COL_HN = 129, 161
COL_VE, COL_VN = 162, 194
COL_FE, COL_FN = 195, 227

_NORM_DIV = float(np.sqrt(np.float32(1.0 / 12.0 + 1e-7)))

_mesh = plsc.VectorSubcoreMesh(core_axis_name="c", subcore_axis_name="s")


@functools.partial(
    pl.kernel,
    out_type=jax.ShapeDtypeStruct((B, OUT_D), jnp.float32),
    mesh=_mesh,
    compiler_params=pltpu.CompilerParams(use_tc_tiling_on_sc=False,
                                         needs_layout_passes=False),
    scratch_types=[
        pltpu.VMEM((BPW,), jnp.int32),
        pltpu.VMEM((BPW,), jnp.int32),
        pltpu.VMEM((BPW,), jnp.int32),
        pltpu.VMEM((BPW,), jnp.float32),
        pltpu.VMEM((BPW,), jnp.float32),
        pltpu.VMEM((BPW,), jnp.float32),
        pltpu.VMEM((BPW,), jnp.float32),
        pltpu.VMEM((NUM_BOUND,), jnp.float32),
        pltpu.VMEM((BPW,), jnp.int32),
        pltpu.VMEM((BPW,), jnp.int32),
        pltpu.VMEM((BPW,), jnp.int32),
        pltpu.VMEM((BPW,), jnp.int32),
        pltpu.VMEM((C, OUT_D), jnp.float32),
        pltpu.VMEM((C, EMBED_DIM), jnp.float32),
        pltpu.VMEM((C, EMBED_DIM), jnp.float32),
        pltpu.VMEM((C, EMBED_DIM), jnp.float32),
        pltpu.VMEM((C, EMBED_DIM), jnp.float32),
        pltpu.VMEM((C, EMBED_DIM), jnp.float32),
        pltpu.VMEM((C, EMBED_DIM), jnp.float32),
        pltpu.VMEM((C, EMBED_DIM), jnp.float32),
        pltpu.SemaphoreType.DMA,
    ],
)
def _sc_kernel(uid_hbm, gid_hbm, sid_hbm, f0_hbm, f1_hbm, f2_hbm, f3_hbm,
               ut_hbm, gt_hbm, st_hbm, rt_hbm, ht_hbm, vt_hbm, ft_hbm,
               bnd_hbm, out_hbm,
               uid_v, gid_v, sid_v, f0_v, f1_v, f2_v, f3_v, bnd_v,
               b0_v, b1_v, b2_v, b3_v, stage_v,
               ue_v, ge_v, se_v, re_v, he_v, ve_v, fe_v, sem):
    wid = lax.axis_index("s") * NC + lax.axis_index("c")
    base = wid * BPW

    with jax.named_scope("init_copies"):
        pltpu.sync_copy(uid_hbm.at[pl.ds(base, BPW)], uid_v)
        pltpu.sync_copy(gid_hbm.at[pl.ds(base, BPW)], gid_v)
        pltpu.sync_copy(sid_hbm.at[pl.ds(base, BPW)], sid_v)
        pltpu.sync_copy(f0_hbm.at[pl.ds(base, BPW)], f0_v)
        pltpu.sync_copy(f1_hbm.at[pl.ds(base, BPW)], f1_v)
        pltpu.sync_copy(f2_hbm.at[pl.ds(base, BPW)], f2_v)
        pltpu.sync_copy(f3_hbm.at[pl.ds(base, BPW)], f3_v)
        pltpu.sync_copy(bnd_hbm, bnd_v)

    feats = ((f0_v, b0_v), (f1_v, b1_v), (f2_v, b2_v), (f3_v, b3_v))

    def bidx_body(i, carry):
        sl = pl.ds(i * L, L)
        for fref, bref in feats:
            x = fref[sl]
            c = jnp.clip((x * 999.0).astype(jnp.int32), 0, NUM_BOUND - 1)
            cm1 = jnp.maximum(c - 1, 0)
            cp1 = jnp.minimum(c + 1, NUM_BOUND - 1)
            t0 = (plsc.load_gather(bnd_v, [cm1]) < x).astype(jnp.int32)
            t1 = (plsc.load_gather(bnd_v, [c]) < x).astype(jnp.int32)
            t2 = (plsc.load_gather(bnd_v, [cp1]) < x).astype(jnp.int32)
            bref[sl] = jnp.maximum(c - 1 + t0 + t1 + t2, 0)
        return carry

    with jax.named_scope("bidx"):
        lax.fori_loop(0, BPW // L, bidx_body, 0)

    lanes = jnp.arange(L, dtype=jnp.int32)

    for ch in range(NCHUNK):
        rbase = ch * C
        csl = pl.ds(rbase, C)
        copies = [
            pltpu.async_copy(ut_hbm.at[uid_v.at[csl]], ue_v, sem),
            pltpu.async_copy(gt_hbm.at[gid_v.at[csl]], ge_v, sem),
            pltpu.async_copy(st_hbm.at[sid_v.at[csl]], se_v, sem),
            pltpu.async_copy(rt_hbm.at[b0_v.at[csl]], re_v, sem),
            pltpu.async_copy(ht_hbm.at[b1_v.at[csl]], he_v, sem),
            pltpu.async_copy(vt_hbm.at[b2_v.at[csl]], ve_v, sem),
            pltpu.async_copy(ft_hbm.at[b3_v.at[csl]], fe_v, sem),
        ]
        with jax.named_scope("gather_wait"):
            for cp in copies:
                cp.wait()

        fields = ((ue_v, COL_U), (ge_v, COL_G), (se_v, COL_S), (re_v, COL_RE),
                  (he_v, COL_HE), (ve_v, COL_VE), (fe_v, COL_FE))

        def asm_body(i, carry):
            rows = lanes + i * L
            for src_ref, col0 in fields:
                for j in range(EMBED_DIM):
                    jv = jnp.full((L,), j, jnp.int32)
                    x = plsc.load_gather(src_ref, [rows, jv])
                    plsc.store_scatter(stage_v, [rows, jv + col0], x)
            for fref, col in ((f0_v, COL_RN), (f1_v, COL_HN),
                              (f2_v, COL_VN), (f3_v, COL_FN)):
                x = fref[pl.ds(rbase + i * L, L)]
                n = (x - 0.5) / _NORM_DIV
                plsc.store_scatter(
                    stage_v, [rows, jnp.full((L,), col, jnp.int32)], n)
            return carry

        with jax.named_scope("asm"):
            lax.fori_loop(0, C // L, asm_body, 0)
        with jax.named_scope("out_write"):
            pltpu.sync_copy(stage_v, out_hbm.at[pl.ds(base + rbase, C), :])


def kernel(user_id, gender, status, regis_date, history, voting, favourite,
           user_table, gender_table, status_table,
           rgst_table, hsty_table, vote_table, favr_table):
    bounds = jnp.linspace(0.0, 1.0, NUM_BOUND)
    return _sc_kernel(
        user_id.astype(jnp.int32), gender.astype(jnp.int32),
        status.astype(jnp.int32), regis_date, history, voting, favourite,
        user_table, gender_table, status_table,
        rgst_table, hsty_table, vote_table, favr_table,
        bounds.astype(jnp.float32))


# speedup vs baseline: 5.8782x; 1.0766x over previous
import functools

import numpy as np
import jax
import jax.numpy as jnp
from jax import lax
from jax.experimental import pallas as pl
from jax.experimental.pallas import tpu as pltpu
from jax.experimental.pallas import tpu_sc as plsc

B = 16384
EMBED_DIM = 32
NUM_BOUND = 1000
OUT_D = 7 * EMBED_DIM + 4

NC, NS, L = 2, 16, 16
NW = NC * NS
BPW = B // NW
C = 256
NCHUNK = BPW // C
NSPLIT = 2

COL_U, COL_G, COL_S, COL_RE = 0, 32, 64, 96
COL_RN = 128
COL_HE, COL_HN = 129, 161
COL_VE, COL_VN = 162, 194
COL_FE, COL_FN = 195, 227

_NORM_DIV = float(np.sqrt(np.float32(1.0 / 12.0 + 1e-7)))

_mesh = plsc.VectorSubcoreMesh(core_axis_name="c", subcore_axis_name="s")


@functools.partial(
    pl.kernel,
    out_type=jax.ShapeDtypeStruct((B, OUT_D), jnp.float32),
    mesh=_mesh,
    compiler_params=pltpu.CompilerParams(use_tc_tiling_on_sc=False,
                                         needs_layout_passes=False),
    scratch_types=[
        pltpu.VMEM((BPW,), jnp.int32),
        pltpu.VMEM((BPW,), jnp.int32),
        pltpu.VMEM((BPW,), jnp.int32),
        pltpu.VMEM((BPW,), jnp.float32),
        pltpu.VMEM((BPW,), jnp.float32),
        pltpu.VMEM((BPW,), jnp.float32),
        pltpu.VMEM((BPW,), jnp.float32),
        pltpu.VMEM((NUM_BOUND,), jnp.float32),
        pltpu.VMEM((BPW,), jnp.int32),
        pltpu.VMEM((BPW,), jnp.int32),
        pltpu.VMEM((BPW,), jnp.int32),
        pltpu.VMEM((BPW,), jnp.int32),
        pltpu.VMEM((C, OUT_D), jnp.float32),
        pltpu.VMEM((C, EMBED_DIM), jnp.float32),
        pltpu.VMEM((C, EMBED_DIM), jnp.float32),
        pltpu.VMEM((C, EMBED_DIM), jnp.float32),
        pltpu.VMEM((C, EMBED_DIM), jnp.float32),
        pltpu.VMEM((C, EMBED_DIM), jnp.float32),
        pltpu.VMEM((C, EMBED_DIM), jnp.float32),
        pltpu.VMEM((C, EMBED_DIM), jnp.float32),
        pltpu.SemaphoreType.DMA,
    ],
)
def _sc_kernel(uid_hbm, gid_hbm, sid_hbm, f0_hbm, f1_hbm, f2_hbm, f3_hbm,
               ut_hbm, gt_hbm, st_hbm, rt_hbm, ht_hbm, vt_hbm, ft_hbm,
               bnd_hbm, out_hbm,
               uid_v, gid_v, sid_v, f0_v, f1_v, f2_v, f3_v, bnd_v,
               b0_v, b1_v, b2_v, b3_v, stage_v,
               ue_v, ge_v, se_v, re_v, he_v, ve_v, fe_v, sem):
    wid = lax.axis_index("s") * NC + lax.axis_index("c")
    base = wid * BPW

    with jax.named_scope("init_copies"):
        pltpu.sync_copy(uid_hbm.at[pl.ds(base, BPW)], uid_v)
        pltpu.sync_copy(gid_hbm.at[pl.ds(base, BPW)], gid_v)
        pltpu.sync_copy(sid_hbm.at[pl.ds(base, BPW)], sid_v)
        pltpu.sync_copy(f0_hbm.at[pl.ds(base, BPW)], f0_v)
        pltpu.sync_copy(f1_hbm.at[pl.ds(base, BPW)], f1_v)
        pltpu.sync_copy(f2_hbm.at[pl.ds(base, BPW)], f2_v)
        pltpu.sync_copy(f3_hbm.at[pl.ds(base, BPW)], f3_v)
        pltpu.sync_copy(bnd_hbm, bnd_v)

    feats = ((f0_v, b0_v), (f1_v, b1_v), (f2_v, b2_v), (f3_v, b3_v))

    def bidx_body(i, carry):
        sl = pl.ds(i * L, L)
        for fref, bref in feats:
            x = fref[sl]
            c = jnp.clip((x * 999.0).astype(jnp.int32), 0, NUM_BOUND - 1)
            cm1 = jnp.maximum(c - 1, 0)
            cp1 = jnp.minimum(c + 1, NUM_BOUND - 1)
            t0 = (plsc.load_gather(bnd_v, [cm1]) < x).astype(jnp.int32)
            t1 = (plsc.load_gather(bnd_v, [c]) < x).astype(jnp.int32)
            t2 = (plsc.load_gather(bnd_v, [cp1]) < x).astype(jnp.int32)
            bref[sl] = jnp.maximum(c - 1 + t0 + t1 + t2, 0)
        return carry

    with jax.named_scope("bidx"):
        lax.fori_loop(0, BPW // L, bidx_body, 0)

    lanes = jnp.arange(L, dtype=jnp.int32)

    srcs = ((ut_hbm, uid_v, ue_v), (gt_hbm, gid_v, ge_v),
            (st_hbm, sid_v, se_v), (rt_hbm, b0_v, re_v),
            (ht_hbm, b1_v, he_v), (vt_hbm, b2_v, ve_v),
            (ft_hbm, b3_v, fe_v))
    fields = ((ue_v, COL_U), (ge_v, COL_G), (se_v, COL_S), (re_v, COL_RE),
              (he_v, COL_HE), (ve_v, COL_VE), (fe_v, COL_FE))
    CS = C // NSPLIT

    for ch in range(NCHUNK):
        rbase = ch * C
        copies = [
            pltpu.async_copy(
                tab.at[idx.at[pl.ds(rbase + sp * CS, CS)]],
                dst.at[pl.ds(sp * CS, CS)], sem)
            for tab, idx, dst in srcs for sp in range(NSPLIT)
        ]

        def norm_body(i, carry):
            rows = lanes + i * L
            for fref, col in ((f0_v, COL_RN), (f1_v, COL_HN),
                              (f2_v, COL_VN), (f3_v, COL_FN)):
                x = fref[pl.ds(rbase + i * L, L)]
                n = (x - 0.5) / _NORM_DIV
                plsc.store_scatter(
                    stage_v, [rows, jnp.full((L,), col, jnp.int32)], n)
            return carry

        with jax.named_scope("norm"):
            lax.fori_loop(0, C // L, norm_body, 0)

        with jax.named_scope("gather_wait"):
            for cp in copies:
                cp.wait()

        src_lo = lanes
        src_hi = lanes + L
        dst_cols = [(col0 + lanes, col0 + lanes + L) for _, col0 in fields]

        def asm_row(r, carry):
            rv = jnp.full((L,), r, jnp.int32)
            for (src_ref, _), (clo, chi) in zip(fields, dst_cols):
                x = plsc.load_gather(src_ref, [rv, src_lo])
                plsc.store_scatter(stage_v, [rv, clo], x)
                y = plsc.load_gather(src_ref, [rv, src_hi])
                plsc.store_scatter(stage_v, [rv, chi], y)
            return carry

        with jax.named_scope("asm"):
            lax.fori_loop(0, C, asm_row, 0)
        with jax.named_scope("out_write"):
            pltpu.sync_copy(stage_v, out_hbm.at[pl.ds(base + rbase, C), :])


def kernel(user_id, gender, status, regis_date, history, voting, favourite,
           user_table, gender_table, status_table,
           rgst_table, hsty_table, vote_table, favr_table):
    bounds = jnp.linspace(0.0, 1.0, NUM_BOUND)
    return _sc_kernel(
        user_id.astype(jnp.int32), gender.astype(jnp.int32),
        status.astype(jnp.int32), regis_date, history, voting, favourite,
        user_table, gender_table, status_table,
        rgst_table, hsty_table, vote_table, favr_table,
        bounds.astype(jnp.float32))


# speedup vs baseline: 7.2785x; 1.2382x over previous
import functools

import numpy as np
import jax
import jax.numpy as jnp
from jax import lax
from jax.experimental import pallas as pl
from jax.experimental.pallas import tpu as pltpu
from jax.experimental.pallas import tpu_sc as plsc

B = 16384
EMBED_DIM = 32
NUM_BOUND = 1000
OUT_D = 7 * EMBED_DIM + 4

NC, NS, L = 2, 16, 16
NW = NC * NS
BPW = B // NW
C = 256
NCHUNK = BPW // C
NSPLIT = 2

COL_U, COL_G, COL_S, COL_RE = 0, 32, 64, 96
COL_RN = 128
COL_HE, COL_HN = 161 - EMBED_DIM, 161
COL_VE, COL_VN = 194 - EMBED_DIM, 194
COL_FE, COL_FN = 227 - EMBED_DIM, 227

_NORM_DIV = float(np.sqrt(np.float32(1.0 / 12.0 + 1e-7)))

_mesh = plsc.VectorSubcoreMesh(core_axis_name="c", subcore_axis_name="s")


@functools.partial(
    pl.kernel,
    out_type=jax.ShapeDtypeStruct((B, OUT_D), jnp.float32),
    mesh=_mesh,
    compiler_params=pltpu.CompilerParams(use_tc_tiling_on_sc=False,
                                         needs_layout_passes=False),
    scratch_types=[
        pltpu.VMEM((BPW,), jnp.int32),
        pltpu.VMEM((BPW,), jnp.int32),
        pltpu.VMEM((BPW,), jnp.int32),
        pltpu.VMEM((BPW,), jnp.float32),
        pltpu.VMEM((BPW,), jnp.float32),
        pltpu.VMEM((BPW,), jnp.float32),
        pltpu.VMEM((BPW,), jnp.float32),
        pltpu.VMEM((NUM_BOUND,), jnp.float32),
        pltpu.VMEM((BPW,), jnp.int32),
        pltpu.VMEM((BPW,), jnp.int32),
        pltpu.VMEM((BPW,), jnp.int32),
        pltpu.VMEM((BPW,), jnp.int32),
        pltpu.VMEM((C, OUT_D), jnp.float32),
        pltpu.VMEM((BPW, EMBED_DIM), jnp.float32),
        pltpu.VMEM((C, EMBED_DIM), jnp.float32),
        pltpu.VMEM((C, EMBED_DIM), jnp.float32),
        pltpu.VMEM((C, EMBED_DIM), jnp.float32),
        pltpu.VMEM((C, EMBED_DIM), jnp.float32),
        pltpu.VMEM((3, EMBED_DIM), jnp.float32),
        pltpu.VMEM((8, EMBED_DIM), jnp.float32),
        pltpu.VMEM_SHARED((NUM_BOUND + 1, EMBED_DIM), jnp.float32),
        pltpu.VMEM_SHARED((NUM_BOUND + 1, EMBED_DIM), jnp.float32),
        pltpu.VMEM_SHARED((NUM_BOUND + 1, EMBED_DIM), jnp.float32),
        pltpu.VMEM_SHARED((NUM_BOUND + 1, EMBED_DIM), jnp.float32),
        pltpu.SemaphoreType.DMA,
        pltpu.SemaphoreType.DMA,
        pltpu.SemaphoreType.DMA,
    ],
)
def _sc_kernel(uid_hbm, gid_hbm, sid_hbm, f0_hbm, f1_hbm, f2_hbm, f3_hbm,
               ut_hbm, gt_hbm, st_hbm, rt_hbm, ht_hbm, vt_hbm, ft_hbm,
               bnd_hbm, out_hbm,
               uid_v, gid_v, sid_v, f0_v, f1_v, f2_v, f3_v, bnd_v,
               b0_v, b1_v, b2_v, b3_v, stage_v,
               ue_v, re_v, he_v, ve_v, fe_v, gt_l, st_l,
               rt_sp, ht_sp, vt_sp, ft_sp, sem, sem_u0, sem_u1):
    sid_ax = lax.axis_index("s")
    wid = sid_ax * NC + lax.axis_index("c")
    base = wid * BPW

    @pl.when(sid_ax == 0)
    def _():
        pltpu.sync_copy(rt_hbm, rt_sp)
        pltpu.sync_copy(ht_hbm, ht_sp)
        pltpu.sync_copy(vt_hbm, vt_sp)
        pltpu.sync_copy(ft_hbm, ft_sp)

    with jax.named_scope("init_copies"):
        pltpu.sync_copy(uid_hbm.at[pl.ds(base, BPW)], uid_v)
        pltpu.sync_copy(gid_hbm.at[pl.ds(base, BPW)], gid_v)
        pltpu.sync_copy(sid_hbm.at[pl.ds(base, BPW)], sid_v)
        pltpu.sync_copy(f0_hbm.at[pl.ds(base, BPW)], f0_v)
        pltpu.sync_copy(f1_hbm.at[pl.ds(base, BPW)], f1_v)
        pltpu.sync_copy(f2_hbm.at[pl.ds(base, BPW)], f2_v)
        pltpu.sync_copy(f3_hbm.at[pl.ds(base, BPW)], f3_v)
        pltpu.sync_copy(bnd_hbm, bnd_v)
        pltpu.sync_copy(gt_hbm, gt_l)
        pltpu.sync_copy(st_hbm, st_l)

    CS = C // NSPLIT
    user_sems = (sem_u0, sem_u1)
    user_copies = [[
        pltpu.async_copy(
            ut_hbm.at[uid_v.at[pl.ds(ch * C + sp * CS, CS)]],
            ue_v.at[pl.ds(ch * C + sp * CS, CS)], user_sems[ch])
        for sp in range(NSPLIT)] for ch in range(NCHUNK)]

    feats = ((f0_v, b0_v), (f1_v, b1_v), (f2_v, b2_v), (f3_v, b3_v))

    def bidx_body(i, carry):
        sl = pl.ds(i * L, L)
        for fref, bref in feats:
            x = fref[sl]
            c = jnp.clip((x * 999.0).astype(jnp.int32), 0, NUM_BOUND - 1)
            cm1 = jnp.maximum(c - 1, 0)
            cp1 = jnp.minimum(c + 1, NUM_BOUND - 1)
            t0 = (plsc.load_gather(bnd_v, [cm1]) < x).astype(jnp.int32)
            t1 = (plsc.load_gather(bnd_v, [c]) < x).astype(jnp.int32)
            t2 = (plsc.load_gather(bnd_v, [cp1]) < x).astype(jnp.int32)
            bref[sl] = jnp.maximum(c - 1 + t0 + t1 + t2, 0)
        return carry

    with jax.named_scope("bidx"):
        lax.fori_loop(0, BPW // L, bidx_body, 0)

    plsc.subcore_barrier()

    lanes = jnp.arange(L, dtype=jnp.int32)
    src_lo = lanes
    src_hi = lanes + L

    bsrcs = ((rt_sp, b0_v, re_v), (ht_sp, b1_v, he_v),
             (vt_sp, b2_v, ve_v), (ft_sp, b3_v, fe_v))
    fields = ((re_v, COL_RE), (he_v, COL_HE), (ve_v, COL_VE), (fe_v, COL_FE))
    dst_cols = [(col0 + lanes, col0 + lanes + L) for _, col0 in fields]
    u_cols = (COL_U + lanes, COL_U + lanes + L)
    g_cols = (COL_G + lanes, COL_G + lanes + L)
    s_cols = (COL_S + lanes, COL_S + lanes + L)

    for ch in range(NCHUNK):
        rbase = ch * C
        csl = pl.ds(rbase, C)
        bcopies = [pltpu.async_copy(tab.at[idx.at[csl]], dst, sem)
                   for tab, idx, dst in bsrcs]

        def norm_body(i, carry):
            rows = lanes + i * L
            for fref, col in ((f0_v, COL_RN), (f1_v, COL_HN),
                              (f2_v, COL_VN), (f3_v, COL_FN)):
                x = fref[pl.ds(rbase + i * L, L)]
                n = (x - 0.5) / _NORM_DIV
                plsc.store_scatter(
                    stage_v, [rows, jnp.full((L,), col, jnp.int32)], n)
            return carry

        with jax.named_scope("norm"):
            lax.fori_loop(0, C // L, norm_body, 0)

        with jax.named_scope("gather_wait"):
            for cp in bcopies:
                cp.wait()
            for cp in user_copies[ch]:
                cp.wait()

        def asm_row(r, carry):
            rv = jnp.full((L,), r, jnp.int32)
            rgv = rv + rbase
            x = plsc.load_gather(ue_v, [rgv, src_lo])
            plsc.store_scatter(stage_v, [rv, u_cols[0]], x)
            y = plsc.load_gather(ue_v, [rgv, src_hi])
            plsc.store_scatter(stage_v, [rv, u_cols[1]], y)
            gv = plsc.load_gather(gid_v, [rgv])
            x = plsc.load_gather(gt_l, [gv, src_lo])
            plsc.store_scatter(stage_v, [rv, g_cols[0]], x)
            y = plsc.load_gather(gt_l, [gv, src_hi])
            plsc.store_scatter(stage_v, [rv, g_cols[1]], y)
            sv = plsc.load_gather(sid_v, [rgv])
            x = plsc.load_gather(st_l, [sv, src_lo])
            plsc.store_scatter(stage_v, [rv, s_cols[0]], x)
            y = plsc.load_gather(st_l, [sv, src_hi])
            plsc.store_scatter(stage_v, [rv, s_cols[1]], y)
            for (src_ref, _), (clo, chi) in zip(fields, dst_cols):
                x = plsc.load_gather(src_ref, [rv, src_lo])
                plsc.store_scatter(stage_v, [rv, clo], x)
                y = plsc.load_gather(src_ref, [rv, src_hi])
                plsc.store_scatter(stage_v, [rv, chi], y)
            return carry

        with jax.named_scope("asm"):
            lax.fori_loop(0, C, asm_row, 0)
        with jax.named_scope("out_write"):
            pltpu.sync_copy(stage_v, out_hbm.at[pl.ds(base + rbase, C), :])


def kernel(user_id, gender, status, regis_date, history, voting, favourite,
           user_table, gender_table, status_table,
           rgst_table, hsty_table, vote_table, favr_table):
    bounds = jnp.linspace(0.0, 1.0, NUM_BOUND)
    return _sc_kernel(
        user_id.astype(jnp.int32), gender.astype(jnp.int32),
        status.astype(jnp.int32), regis_date, history, voting, favourite,
        user_table, gender_table, status_table,
        rgst_table, hsty_table, vote_table, favr_table,
        bounds.astype(jnp.float32))
